# Initial kernel scaffold; baseline (speedup 1.0000x reference)
import jax
import jax.numpy as jnp
from jax.experimental import pallas as pl


def kernel(x, scope_positions, scope_depths, abs_table, scope_table, depth_table, W):
    raise NotImplementedError("write your pallas kernel here")


# speedup vs baseline: 1.4703x; 1.4703x over previous
import functools

import jax
import jax.numpy as jnp
from jax import lax
from jax.experimental import pallas as pl
from jax.experimental.pallas import tpu as pltpu
from jax.experimental.pallas import tpu_sc as plsc

_B, _T, _HIDDEN = 4, 4096, 2048
_SRPE = 256
_ABS_DIM = _HIDDEN - _SRPE
_HALF = _SRPE // 2
_M = _B * _T

_CH = 128


@functools.cache
def _sc_gather():
    info = plsc.get_sparse_core_info()
    nw = info.num_cores * info.num_subcores
    rows_per_w = _M // nw
    n_ch = rows_per_w // _CH
    mesh = plsc.VectorSubcoreMesh(core_axis_name="c", subcore_axis_name="s")

    @functools.partial(
        pl.kernel,
        out_type=(
            jax.ShapeDtypeStruct((_M, _HALF), jnp.float32),
            jax.ShapeDtypeStruct((_M, _HALF), jnp.float32),
        ),
        mesh=mesh,
        scratch_types=[
            pltpu.VMEM((_CH,), jnp.int32),
            pltpu.VMEM((_CH, _HALF), jnp.float32),
            pltpu.SemaphoreType.DMA,
        ],
    )
    def gather_k(scope_hbm, depth_hbm, sidx_hbm, didx_hbm, se_out, de_out,
                 idx_v, rows_v, sem):
        wid = lax.axis_index("s") * info.num_cores + lax.axis_index("c")
        base = wid * rows_per_w
        for table, idx_hbm, out in (
            (scope_hbm, sidx_hbm, se_out),
            (depth_hbm, didx_hbm, de_out),
        ):
            for c in range(n_ch):
                off = base + c * _CH
                pltpu.sync_copy(idx_hbm.at[pl.ds(off, _CH)], idx_v)
                pltpu.async_copy(table.at[idx_v], rows_v, sem).wait()
                pltpu.sync_copy(rows_v, out.at[pl.ds(off, _CH)])

    return gather_k


def _mm_body(x_ref, abs_ref, se_ref, de_ref, w_ref, o_ref):
    y = x_ref[...] + jnp.concatenate(
        [abs_ref[...], se_ref[...], de_ref[...]], axis=1)
    o_ref[...] = lax.dot_general(
        y, w_ref[...], (((1,), (1,)), ((), ())),
        preferred_element_type=jnp.float32)


@functools.cache
def _mm_call(bm, bn):
    t_blocks = _T // bm
    return pl.pallas_call(
        _mm_body,
        grid=(_M // bm, _HIDDEN // bn),
        in_specs=[
            pl.BlockSpec((bm, _HIDDEN), lambda i, j: (i, 0)),
            pl.BlockSpec((bm, _ABS_DIM), lambda i, j: (i % t_blocks, 0)),
            pl.BlockSpec((bm, _HALF), lambda i, j: (i, 0)),
            pl.BlockSpec((bm, _HALF), lambda i, j: (i, 0)),
            pl.BlockSpec((bn, _HIDDEN), lambda i, j: (j, 0)),
        ],
        out_specs=pl.BlockSpec((bm, bn), lambda i, j: (i, j)),
        out_shape=jax.ShapeDtypeStruct((_M, _HIDDEN), jnp.float32),
        compiler_params=pltpu.CompilerParams(
            dimension_semantics=("arbitrary", "arbitrary")),
    )


@jax.jit
def kernel(x, scope_positions, scope_depths, abs_table, scope_table,
           depth_table, W):
    sidx = scope_positions.reshape(_M).astype(jnp.int32)
    didx = scope_depths.reshape(_M).astype(jnp.int32)
    se, de = _sc_gather()(scope_table, depth_table, sidx, didx)
    out = _mm_call(1024, 512)(
        x.reshape(_M, _HIDDEN), abs_table[:_T], se, de, W)
    return out.reshape(_B, _T, _HIDDEN)


# speedup vs baseline: 1.6067x; 1.0928x over previous
import functools

import jax
import jax.numpy as jnp
from jax import lax
from jax.experimental import pallas as pl
from jax.experimental.pallas import tpu as pltpu
from jax.experimental.pallas import tpu_sc as plsc

_B, _T, _HIDDEN = 4, 4096, 2048
_SRPE = 256
_ABS_DIM = _HIDDEN - _SRPE
_HALF = _SRPE // 2
_M = _B * _T

_CH = 128


@functools.cache
def _sc_gather():
    info = plsc.get_sparse_core_info()
    nw = info.num_cores * info.num_subcores
    rows_per_w = _M // nw
    n_ch = rows_per_w // _CH
    mesh = plsc.VectorSubcoreMesh(core_axis_name="c", subcore_axis_name="s")

    @functools.partial(
        pl.kernel,
        out_type=(
            jax.ShapeDtypeStruct((_M, _HALF), jnp.float32),
            jax.ShapeDtypeStruct((_M, _HALF), jnp.float32),
        ),
        mesh=mesh,
        scratch_types=[
            pltpu.VMEM((_CH,), jnp.int32),
            pltpu.VMEM((_CH, _HALF), jnp.float32),
            pltpu.SemaphoreType.DMA,
        ],
    )
    def gather_k(scope_hbm, depth_hbm, sidx_hbm, didx_hbm, se_out, de_out,
                 idx_v, rows_v, sem):
        wid = lax.axis_index("s") * info.num_cores + lax.axis_index("c")
        base = wid * rows_per_w
        for table, idx_hbm, out in (
            (scope_hbm, sidx_hbm, se_out),
            (depth_hbm, didx_hbm, de_out),
        ):
            for c in range(n_ch):
                off = base + c * _CH
                pltpu.sync_copy(idx_hbm.at[pl.ds(off, _CH)], idx_v)
                pltpu.async_copy(table.at[idx_v], rows_v, sem).wait()
                pltpu.sync_copy(rows_v, out.at[pl.ds(off, _CH)])

    return gather_k


def _mm_body(x_ref, abs_ref, se_ref, de_ref, w_ref, o_ref):
    emb = jnp.concatenate(
        [abs_ref[...].astype(jnp.float32), se_ref[...], de_ref[...]], axis=1)
    y = (x_ref[...] + emb).astype(jnp.bfloat16)
    o_ref[...] = lax.dot_general(
        y, w_ref[...], (((1,), (1,)), ((), ())),
        preferred_element_type=jnp.float32)


@functools.cache
def _mm_call(bm, bn):
    t_blocks = _T // bm
    return pl.pallas_call(
        _mm_body,
        grid=(_M // bm, _HIDDEN // bn),
        in_specs=[
            pl.BlockSpec((bm, _HIDDEN), lambda i, j: (i, 0)),
            pl.BlockSpec((bm, _ABS_DIM), lambda i, j: (i % t_blocks, 0)),
            pl.BlockSpec((bm, _HALF), lambda i, j: (i, 0)),
            pl.BlockSpec((bm, _HALF), lambda i, j: (i, 0)),
            pl.BlockSpec((bn, _HIDDEN), lambda i, j: (j, 0)),
        ],
        out_specs=pl.BlockSpec((bm, bn), lambda i, j: (i, j)),
        out_shape=jax.ShapeDtypeStruct((_M, _HIDDEN), jnp.float32),
        compiler_params=pltpu.CompilerParams(
            dimension_semantics=("arbitrary", "arbitrary")),
    )


@jax.jit
def kernel(x, scope_positions, scope_depths, abs_table, scope_table,
           depth_table, W):
    sidx = scope_positions.reshape(_M).astype(jnp.int32)
    didx = scope_depths.reshape(_M).astype(jnp.int32)
    se, de = _sc_gather()(scope_table, depth_table, sidx, didx)
    out = _mm_call(1024, 512)(
        x.reshape(_M, _HIDDEN), abs_table[:_T].astype(jnp.bfloat16), se, de,
        W.astype(jnp.bfloat16))
    return out.reshape(_B, _T, _HIDDEN)


# speedup vs baseline: 2.1323x; 1.3271x over previous
import functools

import jax
import jax.numpy as jnp
from jax import lax
from jax.experimental import pallas as pl
from jax.experimental.pallas import tpu as pltpu
from jax.experimental.pallas import tpu_sc as plsc

_B, _T, _HIDDEN = 4, 4096, 2048
_SRPE = 256
_ABS_DIM = _HIDDEN - _SRPE
_HALF = _SRPE // 2
_M = _B * _T

_CH = 128


@functools.cache
def _sc_gather():
    info = plsc.get_sparse_core_info()
    nw = info.num_cores * info.num_subcores
    rows_per_w = _M // nw
    n_ch = rows_per_w // _CH
    mesh = plsc.VectorSubcoreMesh(core_axis_name="c", subcore_axis_name="s")

    @functools.partial(
        pl.kernel,
        out_type=(
            jax.ShapeDtypeStruct((_M, _HALF), jnp.float32),
            jax.ShapeDtypeStruct((_M, _HALF), jnp.float32),
        ),
        mesh=mesh,
        scratch_types=[
            pltpu.VMEM((2, _CH), jnp.int32),
            pltpu.VMEM((2, _CH, _HALF), jnp.float32),
            [pltpu.SemaphoreType.DMA] * 2,
            [pltpu.SemaphoreType.DMA] * 2,
            [pltpu.SemaphoreType.DMA] * 2,
        ],
    )
    def gather_k(scope_hbm, depth_hbm, sidx_hbm, didx_hbm, se_out, de_out,
                 idx_v, rows_v, sem_i, sem_g, sem_o):
        wid = lax.axis_index("s") * info.num_cores + lax.axis_index("c")
        base = wid * rows_per_w
        chunks = []
        for idx_hbm, table, out in (
            (sidx_hbm, scope_hbm, se_out),
            (didx_hbm, depth_hbm, de_out),
        ):
            for c in range(n_ch):
                off = base + c * _CH
                chunks.append((idx_hbm.at[pl.ds(off, _CH)], table,
                               out.at[pl.ds(off, _CH)]))
        n = len(chunks)
        d_idx = [None, None]
        d_out = [None, None]
        for b in range(2):
            d_idx[b] = pltpu.async_copy(chunks[b][0], idx_v.at[b], sem_i[b])
        for c in range(n):
            b = c % 2
            d_idx[b].wait()
            if d_out[b] is not None:
                d_out[b].wait()
            pltpu.async_copy(chunks[c][1].at[idx_v.at[b]], rows_v.at[b],
                             sem_g[b]).wait()
            if c + 2 < n:
                d_idx[b] = pltpu.async_copy(chunks[c + 2][0], idx_v.at[b],
                                            sem_i[b])
            d_out[b] = pltpu.async_copy(rows_v.at[b], chunks[c][2], sem_o[b])
        for b in range(2):
            d_out[b].wait()

    return gather_k


def _mm_body(x_ref, abs_ref, se_ref, de_ref, w_ref, o_ref):
    emb = jnp.concatenate(
        [abs_ref[...].astype(jnp.float32), se_ref[...], de_ref[...]], axis=1)
    y = (x_ref[...] + emb).astype(jnp.bfloat16)
    o_ref[...] = lax.dot_general(
        y, w_ref[...], (((1,), (1,)), ((), ())),
        preferred_element_type=jnp.float32)


@functools.cache
def _mm_call(bm):
    t_blocks = _T // bm
    return pl.pallas_call(
        _mm_body,
        grid=(_M // bm,),
        in_specs=[
            pl.BlockSpec((bm, _HIDDEN), lambda i: (i, 0)),
            pl.BlockSpec((bm, _ABS_DIM), lambda i: (i % t_blocks, 0)),
            pl.BlockSpec((bm, _HALF), lambda i: (i, 0)),
            pl.BlockSpec((bm, _HALF), lambda i: (i, 0)),
            pl.BlockSpec((_HIDDEN, _HIDDEN), lambda i: (0, 0)),
        ],
        out_specs=pl.BlockSpec((bm, _HIDDEN), lambda i: (i, 0)),
        out_shape=jax.ShapeDtypeStruct((_M, _HIDDEN), jnp.float32),
        compiler_params=pltpu.CompilerParams(
            dimension_semantics=("arbitrary",)),
    )


@jax.jit
def kernel(x, scope_positions, scope_depths, abs_table, scope_table,
           depth_table, W):
    sidx = scope_positions.reshape(_M).astype(jnp.int32)
    didx = scope_depths.reshape(_M).astype(jnp.int32)
    se, de = _sc_gather()(scope_table, depth_table, sidx, didx)
    out = _mm_call(512)(
        x.reshape(_M, _HIDDEN), abs_table[:_T].astype(jnp.bfloat16), se, de,
        W.astype(jnp.bfloat16))
    return out.reshape(_B, _T, _HIDDEN)


# speedup vs baseline: 2.1698x; 1.0175x over previous
import functools

import jax
import jax.numpy as jnp
from jax import lax
from jax.experimental import pallas as pl
from jax.experimental.pallas import tpu as pltpu
from jax.experimental.pallas import tpu_sc as plsc

_B, _T, _HIDDEN = 4, 4096, 2048
_SRPE = 256
_ABS_DIM = _HIDDEN - _SRPE
_HALF = _SRPE // 2
_M = _B * _T

_CH = 128


@functools.cache
def _sc_gather():
    info = plsc.get_sparse_core_info()
    nw = info.num_cores * info.num_subcores
    rows_per_w = _M // nw
    n_ch = rows_per_w // _CH
    mesh = plsc.VectorSubcoreMesh(core_axis_name="c", subcore_axis_name="s")

    @functools.partial(
        pl.kernel,
        out_type=(
            jax.ShapeDtypeStruct((_M, _HALF), jnp.float32),
            jax.ShapeDtypeStruct((_M, _HALF), jnp.float32),
        ),
        mesh=mesh,
        scratch_types=[
            pltpu.VMEM((2, _CH), jnp.int32),
            pltpu.VMEM((2, _CH, _HALF), jnp.float32),
            [pltpu.SemaphoreType.DMA] * 2,
            [pltpu.SemaphoreType.DMA] * 2,
            [pltpu.SemaphoreType.DMA] * 2,
        ],
    )
    def gather_k(scope_hbm, depth_hbm, sidx_hbm, didx_hbm, se_out, de_out,
                 idx_v, rows_v, sem_i, sem_g, sem_o):
        wid = lax.axis_index("s") * info.num_cores + lax.axis_index("c")
        base = wid * rows_per_w
        chunks = []
        for idx_hbm, table, out in (
            (sidx_hbm, scope_hbm, se_out),
            (didx_hbm, depth_hbm, de_out),
        ):
            for c in range(n_ch):
                off = base + c * _CH
                chunks.append((idx_hbm.at[pl.ds(off, _CH)], table,
                               out.at[pl.ds(off, _CH)]))
        n = len(chunks)
        d_idx = [None, None]
        d_out = [None, None]
        for b in range(2):
            d_idx[b] = pltpu.async_copy(chunks[b][0], idx_v.at[b], sem_i[b])
        for c in range(n):
            b = c % 2
            d_idx[b].wait()
            if d_out[b] is not None:
                d_out[b].wait()
            pltpu.async_copy(chunks[c][1].at[idx_v.at[b]], rows_v.at[b],
                             sem_g[b]).wait()
            if c + 2 < n:
                d_idx[b] = pltpu.async_copy(chunks[c + 2][0], idx_v.at[b],
                                            sem_i[b])
            d_out[b] = pltpu.async_copy(rows_v.at[b], chunks[c][2], sem_o[b])
        for b in range(2):
            d_out[b].wait()

    return gather_k


def _mm_body(x_ref, abs_ref, se_ref, de_ref, w_ref, o_ref):
    emb = jnp.concatenate(
        [abs_ref[...].astype(jnp.float32), se_ref[...], de_ref[...]], axis=1)
    y = (x_ref[...] + emb).astype(jnp.bfloat16)
    o_ref[...] = lax.dot_general(
        y, w_ref[...], (((1,), (0,)), ((), ())),
        preferred_element_type=jnp.float32)


@functools.cache
def _mm_call(bm):
    t_blocks = _T // bm
    return pl.pallas_call(
        _mm_body,
        grid=(_M // bm,),
        in_specs=[
            pl.BlockSpec((bm, _HIDDEN), lambda i: (i, 0)),
            pl.BlockSpec((bm, _ABS_DIM), lambda i: (i % t_blocks, 0)),
            pl.BlockSpec((bm, _HALF), lambda i: (i, 0)),
            pl.BlockSpec((bm, _HALF), lambda i: (i, 0)),
            pl.BlockSpec((_HIDDEN, _HIDDEN), lambda i: (0, 0)),
        ],
        out_specs=pl.BlockSpec((bm, _HIDDEN), lambda i: (i, 0)),
        out_shape=jax.ShapeDtypeStruct((_M, _HIDDEN), jnp.float32),
        compiler_params=pltpu.CompilerParams(
            dimension_semantics=("arbitrary",)),
    )


@jax.jit
def kernel(x, scope_positions, scope_depths, abs_table, scope_table,
           depth_table, W):
    sidx = scope_positions.reshape(_M).astype(jnp.int32)
    didx = scope_depths.reshape(_M).astype(jnp.int32)
    se, de = _sc_gather()(scope_table, depth_table, sidx, didx)
    out = _mm_call(1024)(
        x.reshape(_M, _HIDDEN), abs_table[:_T].astype(jnp.bfloat16), se, de,
        W.astype(jnp.bfloat16).T)
    return out.reshape(_B, _T, _HIDDEN)


# speedup vs baseline: 2.1743x; 1.0021x over previous
import functools

import jax
import jax.numpy as jnp
from jax import lax
from jax.experimental import pallas as pl
from jax.experimental.pallas import tpu as pltpu
from jax.experimental.pallas import tpu_sc as plsc

_B, _T, _HIDDEN = 4, 4096, 2048
_SRPE = 256
_ABS_DIM = _HIDDEN - _SRPE
_HALF = _SRPE // 2
_M = _B * _T

_CH = 128


@functools.cache
def _sc_gather():
    info = plsc.get_sparse_core_info()
    nw = info.num_cores * info.num_subcores
    rows_per_w = _M // nw
    n_ch = rows_per_w // _CH
    mesh = plsc.VectorSubcoreMesh(core_axis_name="c", subcore_axis_name="s")

    @functools.partial(
        pl.kernel,
        out_type=(
            jax.ShapeDtypeStruct((_M, _HALF), jnp.float32),
            jax.ShapeDtypeStruct((_M, _HALF), jnp.float32),
        ),
        mesh=mesh,
        scratch_types=[
            pltpu.VMEM((2, _CH), jnp.int32),
            pltpu.VMEM((2, _CH, _HALF), jnp.float32),
            [pltpu.SemaphoreType.DMA] * 2,
            [pltpu.SemaphoreType.DMA] * 2,
            [pltpu.SemaphoreType.DMA] * 2,
        ],
    )
    def gather_k(scope_hbm, depth_hbm, sidx_hbm, didx_hbm, se_out, de_out,
                 idx_v, rows_v, sem_i, sem_g, sem_o):
        wid = lax.axis_index("s") * info.num_cores + lax.axis_index("c")
        base = wid * rows_per_w
        chunks = []
        for idx_hbm, table, out in (
            (sidx_hbm, scope_hbm, se_out),
            (didx_hbm, depth_hbm, de_out),
        ):
            for c in range(n_ch):
                off = base + c * _CH
                chunks.append((idx_hbm.at[pl.ds(off, _CH)], table,
                               out.at[pl.ds(off, _CH)]))
        n = len(chunks)
        d_idx = [None, None]
        d_out = [None, None]
        for b in range(2):
            d_idx[b] = pltpu.async_copy(chunks[b][0], idx_v.at[b], sem_i[b])
        for c in range(n):
            b = c % 2
            d_idx[b].wait()
            if d_out[b] is not None:
                d_out[b].wait()
            pltpu.async_copy(chunks[c][1].at[idx_v.at[b]], rows_v.at[b],
                             sem_g[b]).wait()
            if c + 2 < n:
                d_idx[b] = pltpu.async_copy(chunks[c + 2][0], idx_v.at[b],
                                            sem_i[b])
            d_out[b] = pltpu.async_copy(rows_v.at[b], chunks[c][2], sem_o[b])
        for b in range(2):
            d_out[b].wait()

    return gather_k


def _mm_body(x_ref, abs_ref, se_ref, de_ref, w_ref, o_ref):
    emb = jnp.concatenate(
        [abs_ref[...].astype(jnp.float32), se_ref[...], de_ref[...]], axis=1)
    y = (x_ref[...] + emb).astype(jnp.bfloat16)
    o_ref[...] = lax.dot_general(
        y, w_ref[...], (((1,), (0,)), ((), ())),
        preferred_element_type=jnp.float32)


@functools.cache
def _mm_call(bm):
    t_blocks = _T // bm
    row = lambda it, ib: (ib * t_blocks + it, 0)
    return pl.pallas_call(
        _mm_body,
        grid=(t_blocks, _B),
        in_specs=[
            pl.BlockSpec((bm, _HIDDEN), row),
            pl.BlockSpec((bm, _ABS_DIM), lambda it, ib: (it, 0)),
            pl.BlockSpec((bm, _HALF), row),
            pl.BlockSpec((bm, _HALF), row),
            pl.BlockSpec((_HIDDEN, _HIDDEN), lambda it, ib: (0, 0)),
        ],
        out_specs=pl.BlockSpec((bm, _HIDDEN), row),
        out_shape=jax.ShapeDtypeStruct((_M, _HIDDEN), jnp.float32),
        compiler_params=pltpu.CompilerParams(
            dimension_semantics=("arbitrary", "arbitrary")),
    )


@jax.jit
def kernel(x, scope_positions, scope_depths, abs_table, scope_table,
           depth_table, W):
    sidx = scope_positions.reshape(_M).astype(jnp.int32)
    didx = scope_depths.reshape(_M).astype(jnp.int32)
    se, de = _sc_gather()(scope_table, depth_table, sidx, didx)
    out = _mm_call(1024)(
        x.reshape(_M, _HIDDEN), abs_table[:_T].astype(jnp.bfloat16), se, de,
        W.astype(jnp.bfloat16).T)
    return out.reshape(_B, _T, _HIDDEN)
